# baseline (speedup 1.0000x reference)
import jax
import jax.numpy as jnp
from jax.experimental import pallas as pl
from jax.experimental.pallas import tpu as pltpu

RING_LEN = 4096
SLOT = 8
IN_DIM = 128
NUM_CLASSES = 10
GAUSS_K = 2
GAUSS_TAU = 0.5
WALK_PROB = 0.2
B, T = 256, 64
H = B // 2
WIN = 2 * GAUSS_K + 1
UNROLL = 16
PRE = 8


def _fwd_kernel(x2_ref, Wp_ref, bp_ref, Wih_ref, Whh_ref, bihT_ref,
                bhhT_ref, WjT_ref, bj_ref, WcT_ref, bcT_ref, w_ref,
                out_ref, hhA_ref, hhB_ref, ihA_ref, ihB_ref,
                giA_ref, giB_ref, xbuf_ref, dsem_ref):
    L = RING_LEN
    HALF = L // 2

    def copy(t):
        return pltpu.make_async_copy(
            x2_ref.at[:, pl.ds(t * IN_DIM, IN_DIM)],
            xbuf_ref.at[t],
            dsem_ref.at[t])

    for t in range(PRE):
        copy(t).start()
    for t in range(T):
        copy(t).wait()
        if t + PRE < T:
            copy(t + PRE).start()
        xt = xbuf_ref[t]
        inp = jnp.dot(xt, Wp_ref[:, :]) + bp_ref[:, :]
        inpT = inp.T
        gi = jnp.dot(Wih_ref[:, :], inpT) + bihT_ref[:, :]
        giA_ref[t] = gi[:, 0:H]
        giB_ref[t] = gi[:, H:B]

    def readout(idx, t, ih_ref, hh_ref):
        idxH = idx + HALF

        def weights(e):
            return jnp.where(e == 0, w_ref[GAUSS_K],
                             jnp.where(e == 1, w_ref[GAUSS_K + 1],
                                       jnp.where(e == 2, w_ref[GAUSS_K + 2],
                                                 0.0)))

        acc = jnp.zeros((SLOT, H), dtype=jnp.float32)
        for s0 in range(0, t, UNROLL):
            n = min(UNROLL, t - s0)
            ih = ih_ref[pl.ds(s0, n), 0, :]
            e = jnp.abs(jnp.bitwise_and(idxH - ih, L - 1) - HALF)
            wt = weights(e)
            hU = hh_ref[pl.ds(s0, n)]
            for k in range(n):
                acc = acc + wt[k:k + 1, :] * hU[k]
        return acc

    def half_step(t, ptr, ih_ref, hh_ref, gi_ref):
        idx = jnp.round(ptr).astype(jnp.int32) % L
        read = readout(idx, t, ih_ref, hh_ref)
        gi = gi_ref[t]
        gh = jnp.dot(Whh_ref[:, :], read) + bhhT_ref[:, :]
        r = jax.nn.sigmoid(gi[0:SLOT] + gh[0:SLOT])
        z = jax.nn.sigmoid(gi[SLOT:2 * SLOT] + gh[SLOT:2 * SLOT])
        n = jnp.tanh(gi[2 * SLOT:3 * SLOT] + r * gh[2 * SLOT:3 * SLOT])
        h = (1.0 - z) * n + z * read
        hh_ref[t] = h
        ih_ref[t] = idx
        target = jax.nn.sigmoid(jnp.dot(WjT_ref[:, :], h) + bj_ref[:, :]) * L
        return ((1.0 - WALK_PROB) * target + WALK_PROB * (ptr + 1.0)) % L

    ptrA = jnp.zeros((1, H), dtype=jnp.float32)
    ptrB = jnp.zeros((1, H), dtype=jnp.float32)
    for t in range(T):
        ptrA = half_step(t, ptrA, ihA_ref, hhA_ref, giA_ref)
        ptrB = half_step(t, ptrB, ihB_ref, hhB_ref, giB_ref)
    idxA = jnp.round(ptrA).astype(jnp.int32) % L
    idxB = jnp.round(ptrB).astype(jnp.int32) % L
    finalA = readout(idxA, T, ihA_ref, hhA_ref)
    finalB = readout(idxB, T, ihB_ref, hhB_ref)
    out_ref[:, 0:H] = jnp.dot(WcT_ref[:, :], finalA) + bcT_ref[:, :]
    out_ref[:, H:B] = jnp.dot(WcT_ref[:, :], finalB) + bcT_ref[:, :]


def kernel(x, Wp, bp, W_ih, W_hh, b_ih, b_hh, Wj, bj, Wc, bc):
    offs = jnp.arange(-GAUSS_K, GAUSS_K + 1)
    w = jnp.exp(-(offs.astype(jnp.float32) ** 2) / (2.0 * GAUSS_TAU ** 2))
    w = w / w.sum()

    x2 = x.reshape(B, T * IN_DIM)
    vmem = pl.BlockSpec(memory_space=pltpu.VMEM)
    smem = pl.BlockSpec(memory_space=pltpu.SMEM)
    any_ = pl.BlockSpec(memory_space=pltpu.MemorySpace.HBM)
    outT = pl.pallas_call(
        _fwd_kernel,
        out_shape=jax.ShapeDtypeStruct((NUM_CLASSES, B), jnp.float32),
        in_specs=[any_] + [vmem] * 10 + [smem],
        out_specs=vmem,
        scratch_shapes=[
            pltpu.VMEM((T, SLOT, H), jnp.float32),
            pltpu.VMEM((T, SLOT, H), jnp.float32),
            pltpu.VMEM((T, 1, H), jnp.int32),
            pltpu.VMEM((T, 1, H), jnp.int32),
            pltpu.VMEM((T, 3 * SLOT, H), jnp.float32),
            pltpu.VMEM((T, 3 * SLOT, H), jnp.float32),
            pltpu.VMEM((T, B, IN_DIM), jnp.float32),
            pltpu.SemaphoreType.DMA((T,)),
        ],
    )(x2, Wp, bp.reshape(1, SLOT), W_ih, W_hh,
      b_ih.reshape(3 * SLOT, 1), b_hh.reshape(3 * SLOT, 1), Wj.T,
      bj.reshape(1, 1), Wc.T, bc.reshape(NUM_CLASSES, 1), w)
    return outT.T


# speedup vs baseline: 1.1597x; 1.1597x over previous
import jax
import jax.numpy as jnp
from jax.experimental import pallas as pl
from jax.experimental.pallas import tpu as pltpu

RING_LEN = 4096
SLOT = 8
IN_DIM = 128
NUM_CLASSES = 10
GAUSS_K = 2
GAUSS_TAU = 0.5
WALK_PROB = 0.2
B, T = 256, 64
H = B // 2
WIN = 2 * GAUSS_K + 1
UNROLL = 16


def _fwd_kernel(xsT_ref, WpT_ref, bpT_ref, Wih_ref, Whh_ref, bihT_ref,
                bhhT_ref, WjT_ref, bj_ref, WcT_ref, bcT_ref, w_ref,
                out_ref, hhA_ref, hhB_ref, ihA_ref, ihB_ref,
                giA_ref, giB_ref):
    L = RING_LEN
    HALF = L // 2

    def project(t, _):
        inp = jnp.dot(WpT_ref[:, :], xsT_ref[t]) + bpT_ref[:, :]
        gi = jnp.dot(Wih_ref[:, :], inp) + bihT_ref[:, :]
        giA_ref[t] = gi[:, 0:H]
        giB_ref[t] = gi[:, H:B]
        return 0

    jax.lax.fori_loop(0, T, project, 0, unroll=4)

    def readout(idx, t, ih_ref, hh_ref):
        idxH = idx + HALF

        def weights(e):
            return jnp.where(e == 0, w_ref[GAUSS_K],
                             jnp.where(e == 1, w_ref[GAUSS_K + 1],
                                       jnp.where(e == 2, w_ref[GAUSS_K + 2],
                                                 0.0)))

        acc = jnp.zeros((SLOT, H), dtype=jnp.float32)
        for s0 in range(0, t, UNROLL):
            n = min(UNROLL, t - s0)
            ih = ih_ref[pl.ds(s0, n), 0, :]
            e = jnp.abs(jnp.bitwise_and(idxH - ih, L - 1) - HALF)
            wt = weights(e)
            hU = hh_ref[pl.ds(s0, n)]
            for k in range(n):
                acc = acc + wt[k:k + 1, :] * hU[k]
        return acc

    def half_step(t, ptr, ih_ref, hh_ref, gi_ref):
        idx = jnp.round(ptr).astype(jnp.int32) % L
        read = readout(idx, t, ih_ref, hh_ref)
        gi = gi_ref[t]
        gh = jnp.dot(Whh_ref[:, :], read) + bhhT_ref[:, :]
        r = jax.nn.sigmoid(gi[0:SLOT] + gh[0:SLOT])
        z = jax.nn.sigmoid(gi[SLOT:2 * SLOT] + gh[SLOT:2 * SLOT])
        n = jnp.tanh(gi[2 * SLOT:3 * SLOT] + r * gh[2 * SLOT:3 * SLOT])
        h = (1.0 - z) * n + z * read
        hh_ref[t] = h
        ih_ref[t] = idx
        target = jax.nn.sigmoid(jnp.dot(WjT_ref[:, :], h) + bj_ref[:, :]) * L
        return ((1.0 - WALK_PROB) * target + WALK_PROB * (ptr + 1.0)) % L

    ptrA = jnp.zeros((1, H), dtype=jnp.float32)
    ptrB = jnp.zeros((1, H), dtype=jnp.float32)
    for t in range(T):
        ptrA = half_step(t, ptrA, ihA_ref, hhA_ref, giA_ref)
        ptrB = half_step(t, ptrB, ihB_ref, hhB_ref, giB_ref)
    idxA = jnp.round(ptrA).astype(jnp.int32) % L
    idxB = jnp.round(ptrB).astype(jnp.int32) % L
    finalA = readout(idxA, T, ihA_ref, hhA_ref)
    finalB = readout(idxB, T, ihB_ref, hhB_ref)
    out_ref[:, 0:H] = jnp.dot(WcT_ref[:, :], finalA) + bcT_ref[:, :]
    out_ref[:, H:B] = jnp.dot(WcT_ref[:, :], finalB) + bcT_ref[:, :]


def kernel(x, Wp, bp, W_ih, W_hh, b_ih, b_hh, Wj, bj, Wc, bc):
    offs = jnp.arange(-GAUSS_K, GAUSS_K + 1)
    w = jnp.exp(-(offs.astype(jnp.float32) ** 2) / (2.0 * GAUSS_TAU ** 2))
    w = w / w.sum()

    xsT = jnp.transpose(x, (1, 2, 0))
    vmem = pl.BlockSpec(memory_space=pltpu.VMEM)
    smem = pl.BlockSpec(memory_space=pltpu.SMEM)
    outT = pl.pallas_call(
        _fwd_kernel,
        out_shape=jax.ShapeDtypeStruct((NUM_CLASSES, B), jnp.float32),
        in_specs=[vmem] * 11 + [smem],
        out_specs=vmem,
        scratch_shapes=[
            pltpu.VMEM((T, SLOT, H), jnp.float32),
            pltpu.VMEM((T, SLOT, H), jnp.float32),
            pltpu.VMEM((T, 1, H), jnp.int32),
            pltpu.VMEM((T, 1, H), jnp.int32),
            pltpu.VMEM((T, 3 * SLOT, H), jnp.float32),
            pltpu.VMEM((T, 3 * SLOT, H), jnp.float32),
        ],
    )(xsT, Wp.T, bp.reshape(SLOT, 1), W_ih, W_hh,
      b_ih.reshape(3 * SLOT, 1), b_hh.reshape(3 * SLOT, 1), Wj.T,
      bj.reshape(1, 1), Wc.T, bc.reshape(NUM_CLASSES, 1), w)
    return outT.T
